# baseline (speedup 1.0000x reference)
import functools

import jax
import jax.numpy as jnp
from jax import lax
from jax.experimental import pallas as pl
from jax.experimental.pallas import tpu as pltpu
from jax.experimental.pallas import tpu_sc as plsc

_N = 10000
_E = 320000
_K = 64
_NSUB = 16
_BN = 1000
_GRID = _N // _BN

_NCHUNK = 5120
_E_PAD = _NCHUNK * _K
_NPAD = _N + 16

_ROWS_MAIN = 632
_ROWS_LAST = _N - (_NSUB - 1) * _ROWS_MAIN

_DEG_CPS = _NCHUNK // 2 // _NSUB
_MP_CPS = _NCHUNK // _NSUB


def _mesh():
    return plsc.VectorSubcoreMesh(core_axis_name="c", subcore_axis_name="s")


@functools.partial(
    pl.kernel,
    out_type=(
        jax.ShapeDtypeStruct((_N,), jnp.float32),
        jax.ShapeDtypeStruct((_N,), jnp.float32),
    ),
    mesh=_mesh(),
    scratch_types=[
        pltpu.VMEM((16, _K), jnp.int32),
        pltpu.VMEM((_K,), jnp.float32),
        pltpu.VMEM((640,), jnp.float32),
        pltpu.VMEM_SHARED((_NPAD,), jnp.float32),
    ],
)
def _deg_kernel(dst2d, out0, out1, didx, ones_v, zeros_v, acc):
    c = lax.axis_index("c")
    s = lax.axis_index("s")
    for i in range(_K // 16):
        ones_v[pl.ds(16 * i, 16)] = jnp.full((16,), 1.0, jnp.float32)
    for i in range(640 // 16):
        zeros_v[pl.ds(16 * i, 16)] = jnp.zeros((16,), jnp.float32)

    half = _NCHUNK // 2
    lo = c * half + s * _DEG_CPS

    @pl.when(s < _NSUB - 1)
    def _():
        pltpu.sync_copy(zeros_v.at[pl.ds(0, _ROWS_MAIN)],
                        acc.at[pl.ds(s * _ROWS_MAIN, _ROWS_MAIN)])

    @pl.when(s == _NSUB - 1)
    def _():
        pltpu.sync_copy(zeros_v.at[pl.ds(0, _ROWS_LAST)],
                        acc.at[pl.ds((_NSUB - 1) * _ROWS_MAIN, _ROWS_LAST)])

    plsc.subcore_barrier()

    def body(j, carry):
        r = lax.rem(j, 16)

        @pl.when(r == 0)
        def _():
            pltpu.sync_copy(dst2d.at[pl.ds(pl.multiple_of(lo + j, 16), 16)], didx)

        pltpu.sync_copy(ones_v, acc.at[didx.at[r]], add=True)
        return carry

    lax.fori_loop(0, _DEG_CPS, body, 0)
    plsc.subcore_barrier()

    def wout(out):
        @pl.when(s < _NSUB - 1)
        def _():
            pltpu.sync_copy(acc.at[pl.ds(s * _ROWS_MAIN, _ROWS_MAIN)],
                            zeros_v.at[pl.ds(0, _ROWS_MAIN)])
            pltpu.sync_copy(zeros_v.at[pl.ds(0, _ROWS_MAIN)],
                            out.at[pl.ds(s * _ROWS_MAIN, _ROWS_MAIN)])

        @pl.when(s == _NSUB - 1)
        def _():
            pltpu.sync_copy(acc.at[pl.ds((_NSUB - 1) * _ROWS_MAIN, _ROWS_LAST)],
                            zeros_v.at[pl.ds(0, _ROWS_LAST)])
            pltpu.sync_copy(zeros_v.at[pl.ds(0, _ROWS_LAST)],
                            out.at[pl.ds((_NSUB - 1) * _ROWS_MAIN, _ROWS_LAST)])

    @pl.when(c == 0)
    def _():
        wout(out0)

    @pl.when(c == 1)
    def _():
        wout(out1)


_CH = 128
_MP_CPS = _NCHUNK // 2 // _NSUB
_NBUF = 4
_PIECE = 16
_NGRP = _MP_CPS // _NBUF


@functools.partial(
    pl.kernel,
    out_type=(
        jax.ShapeDtypeStruct((_N, _CH), jnp.float32),
        jax.ShapeDtypeStruct((_N, _CH), jnp.float32),
    ),
    mesh=_mesh(),
    scratch_types=[
        pltpu.VMEM((3, _PIECE, _K), jnp.int32),
        pltpu.VMEM((3, _PIECE, _K), jnp.int32),
        pltpu.VMEM((_NBUF, _K, _CH), jnp.float32),
        pltpu.VMEM_SHARED((_NPAD, _CH), jnp.float32),
        pltpu.SemaphoreType.DMA((_NBUF,)),
        pltpu.SemaphoreType.DMA((_NBUF,)),
    ],
)
def _mp_kernel(xws, zeros, src2d, dst2d, out0, out1, sidx, didx, rows, acc,
               gsem, ssem):
    c = lax.axis_index("c")
    s = lax.axis_index("s")
    lo = c * (_NCHUNK // 2) + s * _MP_CPS

    def staged(src_at, dst_at, base, total):
        off = 0
        while off < total:
            n = min(_K, total - off)
            pltpu.sync_copy(src_at(pl.ds(base + off, n)),
                            rows.at[0, pl.ds(0, n)])
            pltpu.sync_copy(rows.at[0, pl.ds(0, n)],
                            dst_at(pl.ds(base + off, n)))
            off += n

    @pl.when(s < _NSUB - 1)
    def _():
        staged(lambda d: zeros.at[d], lambda d: acc.at[d],
               s * _ROWS_MAIN, _ROWS_MAIN)

    @pl.when(s == _NSUB - 1)
    def _():
        staged(lambda d: zeros.at[d], lambda d: acc.at[d],
               (_NSUB - 1) * _ROWS_MAIN, _ROWS_LAST)

    pltpu.sync_copy(src2d.at[pl.ds(lo, _PIECE)], sidx.at[0])
    pltpu.sync_copy(dst2d.at[pl.ds(lo, _PIECE)], didx.at[0])
    plsc.subcore_barrier()

    def rs(j):
        return lax.rem(j, _PIECE), lax.rem(lax.div(j, _PIECE), 3)

    def gath(j, b):
        r, slot = rs(j)
        return pltpu.async_copy(xws.at[sidx.at[slot, r]], rows.at[b],
                                gsem.at[b])

    gath(0, 0)
    gath(1, 1)

    def group(jj, carry):
        for b in range(_NBUF):
            j = jj * _NBUF + b

            if b == 0:
                @pl.when(jnp.logical_and(lax.rem(jj, _PIECE // _NBUF) == 0,
                                         jj * _NBUF + _PIECE < _MP_CPS))
                def _():
                    _, slot = rs(jj * _NBUF)
                    nxt = lax.rem(slot + 1, 3)
                    off = pl.multiple_of(lo + jj * _NBUF + _PIECE, _PIECE)
                    pltpu.sync_copy(src2d.at[pl.ds(off, _PIECE)],
                                    sidx.at[nxt])
                    pltpu.sync_copy(dst2d.at[pl.ds(off, _PIECE)],
                                    didx.at[nxt])

            r, slot = rs(j)
            pltpu.make_async_copy(xws.at[sidx.at[slot, r]], rows.at[b],
                                  gsem.at[b]).wait()
            pltpu.async_copy(rows.at[b], acc.at[didx.at[slot, r]],
                             ssem.at[b], add=True)

            bg = (b + 2) % _NBUF

            def recycle():
                pltpu.make_async_copy(rows.at[bg], acc.at[didx.at[0, 0]],
                                      ssem.at[bg]).wait()
                gath(j + 2, bg)

            if b < 2:
                @pl.when(jj > 0)
                def _():
                    recycle()

                @pl.when(jj == 0)
                def _():
                    gath(j + 2, bg)
            else:
                @pl.when(jj < _NGRP - 1)
                def _():
                    recycle()
        return carry

    lax.fori_loop(0, _NGRP, group, 0)
    for b in range(_NBUF):
        pltpu.make_async_copy(rows.at[b], acc.at[didx.at[0, 0]],
                              ssem.at[b]).wait()
    plsc.subcore_barrier()

    def wout(out):
        @pl.when(s < _NSUB - 1)
        def _():
            staged(lambda d: acc.at[d], lambda d: out.at[d],
                   s * _ROWS_MAIN, _ROWS_MAIN)

        @pl.when(s == _NSUB - 1)
        def _():
            staged(lambda d: acc.at[d], lambda d: out.at[d],
                   (_NSUB - 1) * _ROWS_MAIN, _ROWS_LAST)

    @pl.when(c == 0)
    def _():
        wout(out0)

    @pl.when(c == 1)
    def _():
        wout(out1)


def _t1(x, W1, deg0, deg1):
    def body(x_ref, w_ref, d0_ref, d1_ref, h0_ref, h1_ref, dis_ref):
        deg = d0_ref[...] + d1_ref[...] + 1.0
        dis = lax.rsqrt(deg)
        xw = jnp.dot(x_ref[...], w_ref[...], preferred_element_type=jnp.float32)
        xws = xw * dis
        h0_ref[...] = xws[:, :128]
        h1_ref[...] = xws[:, 128:]
        dis_ref[...] = dis

    return pl.pallas_call(
        body,
        grid=(_GRID,),
        in_specs=[
            pl.BlockSpec((_BN, 128), lambda i: (i, 0)),
            pl.BlockSpec((128, 256), lambda i: (0, 0)),
            pl.BlockSpec((_BN, 1), lambda i: (i, 0)),
            pl.BlockSpec((_BN, 1), lambda i: (i, 0)),
        ],
        out_specs=[pl.BlockSpec((_BN, 128), lambda i: (i, 0))] * 2
        + [pl.BlockSpec((_BN, 1), lambda i: (i, 0))],
        out_shape=[jax.ShapeDtypeStruct((_N, 128), jnp.float32)] * 2
        + [jax.ShapeDtypeStruct((_N, 1), jnp.float32)],
    )(x, W1, deg0, deg1)


def _t2(p00, p01, p10, p11, h0, h1, dis, b1, W2):
    def body(p00_ref, p01_ref, p10_ref, p11_ref, h0_ref, h1_ref, dis_ref,
             b_ref, w_ref, o_ref):
        d = dis_ref[...]
        half0 = p00_ref[...] + p01_ref[...] + h0_ref[...]
        half1 = p10_ref[...] + p11_ref[...] + h1_ref[...]
        h = jnp.concatenate([half0, half1], axis=1)
        h = jnp.maximum(h * d + b_ref[...], 0.0)
        xw = jnp.dot(h, w_ref[...], preferred_element_type=jnp.float32)
        o_ref[...] = xw * d

    return pl.pallas_call(
        body,
        grid=(_GRID,),
        in_specs=[pl.BlockSpec((_BN, 128), lambda i: (i, 0))] * 6
        + [
            pl.BlockSpec((_BN, 1), lambda i: (i, 0)),
            pl.BlockSpec((1, 256), lambda i: (0, 0)),
            pl.BlockSpec((256, 128), lambda i: (0, 0)),
        ],
        out_specs=pl.BlockSpec((_BN, 128), lambda i: (i, 0)),
        out_shape=jax.ShapeDtypeStruct((_N, 128), jnp.float32),
    )(p00, p01, p10, p11, h0, h1, dis, b1, W2)


def _t3(p0, p1, xws2, dis, b2, Wl, bl):
    def body(p0_ref, p1_ref, x2_ref, dis_ref, b2_ref, w_ref, bl_ref, o_ref):
        d = dis_ref[...]
        h = (p0_ref[...] + p1_ref[...] + x2_ref[...]) * d + b2_ref[...]
        o = jnp.dot(h, w_ref[...], preferred_element_type=jnp.float32) + bl_ref[...]
        o_ref[...] = jnp.maximum(o, 0.0)

    return pl.pallas_call(
        body,
        grid=(_GRID,),
        in_specs=[pl.BlockSpec((_BN, 128), lambda i: (i, 0))] * 3
        + [
            pl.BlockSpec((_BN, 1), lambda i: (i, 0)),
            pl.BlockSpec((1, 128), lambda i: (0, 0)),
            pl.BlockSpec((128, 128), lambda i: (0, 0)),
            pl.BlockSpec((1, 128), lambda i: (0, 0)),
        ],
        out_specs=pl.BlockSpec((_BN, 128), lambda i: (i, 0)),
        out_shape=jax.ShapeDtypeStruct((_N, 128), jnp.float32),
    )(p0, p1, xws2, dis, b2, Wl, bl)


def kernel(x, edge_index, W1, b1, W2, b2, Wl, bl):
    ei = edge_index.astype(jnp.int32)
    pad = _E_PAD - _E
    padsrc = jnp.arange(pad, dtype=jnp.int32) % _N
    paddst = _N + (jnp.arange(pad, dtype=jnp.int32) % (_NPAD - _N))
    src2d = jnp.concatenate([ei[0], padsrc]).reshape(_NCHUNK, _K)
    dst2d = jnp.concatenate([ei[1], paddst]).reshape(_NCHUNK, _K)
    zeros = jnp.zeros((_N, _CH), jnp.float32)

    d0, d1 = _deg_kernel(dst2d)
    h0, h1, dis = _t1(x, W1, d0.reshape(_N, 1), d1.reshape(_N, 1))
    p00, p01 = _mp_kernel(h0, zeros, src2d, dst2d)
    p10, p11 = _mp_kernel(h1, zeros, src2d, dst2d)
    xws2 = _t2(p00, p01, p10, p11, h0, h1, dis, b1.reshape(1, -1), W2)
    q0, q1 = _mp_kernel(xws2, zeros, src2d, dst2d)
    return _t3(q0, q1, xws2, dis, b2.reshape(1, -1), Wl, bl.reshape(1, -1))


# speedup vs baseline: 1.5944x; 1.5944x over previous
import functools

import jax
import jax.numpy as jnp
from jax import lax
from jax.experimental import pallas as pl
from jax.experimental.pallas import tpu as pltpu
from jax.experimental.pallas import tpu_sc as plsc

_N = 10000
_E = 320000
_K = 128
_NSUB = 16
_BN = 1000
_GRID = _N // _BN

_NCHUNK = 2560
_E_PAD = _NCHUNK * _K
_NPAD = _N + 16

_ROWS_MAIN = 632
_ROWS_LAST = _N - (_NSUB - 1) * _ROWS_MAIN

_DEG_CPS = _NCHUNK // 2 // _NSUB
_MP_CPS = _NCHUNK // _NSUB


def _mesh():
    return plsc.VectorSubcoreMesh(core_axis_name="c", subcore_axis_name="s")


@functools.partial(
    pl.kernel,
    out_type=(
        jax.ShapeDtypeStruct((_N,), jnp.float32),
        jax.ShapeDtypeStruct((_N,), jnp.float32),
    ),
    mesh=_mesh(),
    scratch_types=[
        pltpu.VMEM((16, _K), jnp.int32),
        pltpu.VMEM((_K,), jnp.float32),
        pltpu.VMEM((640,), jnp.float32),
        pltpu.VMEM_SHARED((_NPAD,), jnp.float32),
    ],
)
def _deg_kernel(dst2d, out0, out1, didx, ones_v, zeros_v, acc):
    c = lax.axis_index("c")
    s = lax.axis_index("s")
    for i in range(_K // 16):
        ones_v[pl.ds(16 * i, 16)] = jnp.full((16,), 1.0, jnp.float32)
    for i in range(640 // 16):
        zeros_v[pl.ds(16 * i, 16)] = jnp.zeros((16,), jnp.float32)

    half = _NCHUNK // 2
    lo = c * half + s * _DEG_CPS

    @pl.when(s < _NSUB - 1)
    def _():
        pltpu.sync_copy(zeros_v.at[pl.ds(0, _ROWS_MAIN)],
                        acc.at[pl.ds(s * _ROWS_MAIN, _ROWS_MAIN)])

    @pl.when(s == _NSUB - 1)
    def _():
        pltpu.sync_copy(zeros_v.at[pl.ds(0, _ROWS_LAST)],
                        acc.at[pl.ds((_NSUB - 1) * _ROWS_MAIN, _ROWS_LAST)])

    plsc.subcore_barrier()

    def body(j, carry):
        r = lax.rem(j, 16)

        @pl.when(r == 0)
        def _():
            pltpu.sync_copy(dst2d.at[pl.ds(pl.multiple_of(lo + j, 16), 16)], didx)

        pltpu.sync_copy(ones_v, acc.at[didx.at[r]], add=True)
        return carry

    lax.fori_loop(0, _DEG_CPS, body, 0)
    plsc.subcore_barrier()

    def wout(out):
        @pl.when(s < _NSUB - 1)
        def _():
            pltpu.sync_copy(acc.at[pl.ds(s * _ROWS_MAIN, _ROWS_MAIN)],
                            zeros_v.at[pl.ds(0, _ROWS_MAIN)])
            pltpu.sync_copy(zeros_v.at[pl.ds(0, _ROWS_MAIN)],
                            out.at[pl.ds(s * _ROWS_MAIN, _ROWS_MAIN)])

        @pl.when(s == _NSUB - 1)
        def _():
            pltpu.sync_copy(acc.at[pl.ds((_NSUB - 1) * _ROWS_MAIN, _ROWS_LAST)],
                            zeros_v.at[pl.ds(0, _ROWS_LAST)])
            pltpu.sync_copy(zeros_v.at[pl.ds(0, _ROWS_LAST)],
                            out.at[pl.ds((_NSUB - 1) * _ROWS_MAIN, _ROWS_LAST)])

    @pl.when(c == 0)
    def _():
        wout(out0)

    @pl.when(c == 1)
    def _():
        wout(out1)


_CH = 128
_MP_CPS = _NCHUNK // 2 // _NSUB
_NBUF = 2
_PIECE = 16


@functools.partial(
    pl.kernel,
    out_type=(
        jax.ShapeDtypeStruct((_N, _CH), jnp.float32),
        jax.ShapeDtypeStruct((_N, _CH), jnp.float32),
    ),
    mesh=_mesh(),
    scratch_types=[
        pltpu.VMEM((2, _PIECE, _K), jnp.int32),
        pltpu.VMEM((2, _PIECE, _K), jnp.int32),
        pltpu.VMEM((_NBUF, _K, _CH), jnp.float32),
        pltpu.VMEM_SHARED((_NPAD, _CH), jnp.float32),
        pltpu.SemaphoreType.DMA((_NBUF,)),
    ],
)
def _mp_kernel(xws, zeros, src2d, dst2d, out0, out1, sidx, didx, rows, acc,
               gsem):
    c = lax.axis_index("c")
    s = lax.axis_index("s")
    lo = c * (_NCHUNK // 2) + s * _MP_CPS

    def staged(src_at, dst_at, base, total):
        off = 0
        while off < total:
            n = min(_K, total - off)
            pltpu.sync_copy(src_at(pl.ds(base + off, n)),
                            rows.at[0, pl.ds(0, n)])
            pltpu.sync_copy(rows.at[0, pl.ds(0, n)],
                            dst_at(pl.ds(base + off, n)))
            off += n

    @pl.when(s < _NSUB - 1)
    def _():
        staged(lambda d: zeros.at[d], lambda d: acc.at[d],
               s * _ROWS_MAIN, _ROWS_MAIN)

    @pl.when(s == _NSUB - 1)
    def _():
        staged(lambda d: zeros.at[d], lambda d: acc.at[d],
               (_NSUB - 1) * _ROWS_MAIN, _ROWS_LAST)

    pltpu.sync_copy(src2d.at[pl.ds(lo, _PIECE)], sidx.at[0])
    pltpu.sync_copy(dst2d.at[pl.ds(lo, _PIECE)], didx.at[0])
    plsc.subcore_barrier()

    def prime(j, carry):
        pltpu.async_copy(xws.at[sidx.at[0, j]], rows.at[lax.rem(j, _NBUF)],
                         gsem.at[lax.rem(j, _NBUF)])
        return carry

    lax.fori_loop(0, _NBUF, prime, 0)

    def body(j, carry):
        r = lax.rem(j, _PIECE)
        slot = lax.rem(lax.div(j, _PIECE), 2)

        @pl.when(jnp.logical_and(r == 0, j + _PIECE < _MP_CPS))
        def _():
            nxt = lax.rem(slot + 1, 2)
            off = pl.multiple_of(lo + j + _PIECE, _PIECE)
            pltpu.sync_copy(src2d.at[pl.ds(off, _PIECE)], sidx.at[nxt])
            pltpu.sync_copy(dst2d.at[pl.ds(off, _PIECE)], didx.at[nxt])

        b = lax.rem(j, _NBUF)
        pltpu.make_async_copy(xws.at[sidx.at[slot, r]], rows.at[b],
                              gsem.at[b]).wait()
        pltpu.sync_copy(rows.at[b], acc.at[didx.at[slot, r]], add=True)

        @pl.when(j < _MP_CPS - _NBUF)
        def _():
            j2 = j + _NBUF
            r2 = lax.rem(j2, _PIECE)
            slot2 = lax.rem(lax.div(j2, _PIECE), 2)
            pltpu.async_copy(xws.at[sidx.at[slot2, r2]], rows.at[b],
                             gsem.at[b])
        return carry

    lax.fori_loop(0, _MP_CPS, body, 0)
    plsc.subcore_barrier()

    def wout(out):
        @pl.when(s < _NSUB - 1)
        def _():
            staged(lambda d: acc.at[d], lambda d: out.at[d],
                   s * _ROWS_MAIN, _ROWS_MAIN)

        @pl.when(s == _NSUB - 1)
        def _():
            staged(lambda d: acc.at[d], lambda d: out.at[d],
                   (_NSUB - 1) * _ROWS_MAIN, _ROWS_LAST)

    @pl.when(c == 0)
    def _():
        wout(out0)

    @pl.when(c == 1)
    def _():
        wout(out1)


def _t1(x, deg0, deg1):
    def body(x_ref, d0_ref, d1_ref, xs_ref, dis_ref):
        deg = d0_ref[...] + d1_ref[...] + 1.0
        dis = lax.rsqrt(deg)
        xs_ref[...] = x_ref[...] * dis
        dis_ref[...] = dis

    return pl.pallas_call(
        body,
        grid=(_GRID,),
        in_specs=[
            pl.BlockSpec((_BN, 128), lambda i: (i, 0)),
            pl.BlockSpec((_BN, 1), lambda i: (i, 0)),
            pl.BlockSpec((_BN, 1), lambda i: (i, 0)),
        ],
        out_specs=[
            pl.BlockSpec((_BN, 128), lambda i: (i, 0)),
            pl.BlockSpec((_BN, 1), lambda i: (i, 0)),
        ],
        out_shape=[
            jax.ShapeDtypeStruct((_N, 128), jnp.float32),
            jax.ShapeDtypeStruct((_N, 1), jnp.float32),
        ],
    )(x, deg0, deg1)


def _t2(p0, p1, xs, dis, W1, b1, W2):
    def body(p0_ref, p1_ref, xs_ref, dis_ref, w1_ref, b_ref, w2_ref, o_ref):
        d = dis_ref[...]
        ax = (p0_ref[...] + p1_ref[...] + xs_ref[...]) * d
        h = jnp.dot(ax, w1_ref[...], preferred_element_type=jnp.float32)
        h = jnp.maximum(h + b_ref[...], 0.0)
        xw = jnp.dot(h, w2_ref[...], preferred_element_type=jnp.float32)
        o_ref[...] = xw * d

    return pl.pallas_call(
        body,
        grid=(_GRID,),
        in_specs=[pl.BlockSpec((_BN, 128), lambda i: (i, 0))] * 3
        + [
            pl.BlockSpec((_BN, 1), lambda i: (i, 0)),
            pl.BlockSpec((128, 256), lambda i: (0, 0)),
            pl.BlockSpec((1, 256), lambda i: (0, 0)),
            pl.BlockSpec((256, 128), lambda i: (0, 0)),
        ],
        out_specs=pl.BlockSpec((_BN, 128), lambda i: (i, 0)),
        out_shape=jax.ShapeDtypeStruct((_N, 128), jnp.float32),
    )(p0, p1, xs, dis, W1, b1, W2)


def _t3(p0, p1, xws2, dis, b2, Wl, bl):
    def body(p0_ref, p1_ref, x2_ref, dis_ref, b2_ref, w_ref, bl_ref, o_ref):
        d = dis_ref[...]
        h = (p0_ref[...] + p1_ref[...] + x2_ref[...]) * d + b2_ref[...]
        o = jnp.dot(h, w_ref[...], preferred_element_type=jnp.float32) + bl_ref[...]
        o_ref[...] = jnp.maximum(o, 0.0)

    return pl.pallas_call(
        body,
        grid=(_GRID,),
        in_specs=[pl.BlockSpec((_BN, 128), lambda i: (i, 0))] * 3
        + [
            pl.BlockSpec((_BN, 1), lambda i: (i, 0)),
            pl.BlockSpec((1, 128), lambda i: (0, 0)),
            pl.BlockSpec((128, 128), lambda i: (0, 0)),
            pl.BlockSpec((1, 128), lambda i: (0, 0)),
        ],
        out_specs=pl.BlockSpec((_BN, 128), lambda i: (i, 0)),
        out_shape=jax.ShapeDtypeStruct((_N, 128), jnp.float32),
    )(p0, p1, xws2, dis, b2, Wl, bl)


def kernel(x, edge_index, W1, b1, W2, b2, Wl, bl):
    ei = edge_index.astype(jnp.int32)
    pad = _E_PAD - _E
    padsrc = jnp.arange(pad, dtype=jnp.int32) % _N
    paddst = _N + (jnp.arange(pad, dtype=jnp.int32) % (_NPAD - _N))
    src2d = jnp.concatenate([ei[0], padsrc]).reshape(_NCHUNK, _K)
    dst2d = jnp.concatenate([ei[1], paddst]).reshape(_NCHUNK, _K)
    zeros = jnp.zeros((_N, _CH), jnp.float32)

    d0, d1 = _deg_kernel(dst2d)
    xs, dis = _t1(x, d0.reshape(_N, 1), d1.reshape(_N, 1))
    p0, p1 = _mp_kernel(xs, zeros, src2d, dst2d)
    xws2 = _t2(p0, p1, xs, dis, W1, b1.reshape(1, -1), W2)
    q0, q1 = _mp_kernel(xws2, zeros, src2d, dst2d)
    return _t3(q0, q1, xws2, dis, b2.reshape(1, -1), Wl, bl.reshape(1, -1))


# speedup vs baseline: 1.6271x; 1.0206x over previous
import functools

import jax
import jax.numpy as jnp
from jax import lax
from jax.experimental import pallas as pl
from jax.experimental.pallas import tpu as pltpu
from jax.experimental.pallas import tpu_sc as plsc

_N = 10000
_E = 320000
_K = 128
_NSUB = 16
_BN = 1000
_GRID = _N // _BN

_NCHUNK = 2560
_E_PAD = _NCHUNK * _K
_NPAD = _N + 16

_ROWS_MAIN = 632
_ROWS_LAST = _N - (_NSUB - 1) * _ROWS_MAIN

_DEG_CPS = _NCHUNK // 2 // _NSUB
_MP_CPS = _NCHUNK // _NSUB


def _mesh():
    return plsc.VectorSubcoreMesh(core_axis_name="c", subcore_axis_name="s")


@functools.partial(
    pl.kernel,
    out_type=(
        jax.ShapeDtypeStruct((_N,), jnp.float32),
        jax.ShapeDtypeStruct((_N,), jnp.float32),
    ),
    mesh=_mesh(),
    scratch_types=[
        pltpu.VMEM((16, _K), jnp.int32),
        pltpu.VMEM((_K,), jnp.float32),
        pltpu.VMEM((640,), jnp.float32),
        pltpu.VMEM_SHARED((_NPAD,), jnp.float32),
    ],
)
def _deg_kernel(dst2d, out0, out1, didx, ones_v, zeros_v, acc):
    c = lax.axis_index("c")
    s = lax.axis_index("s")
    for i in range(_K // 16):
        ones_v[pl.ds(16 * i, 16)] = jnp.full((16,), 1.0, jnp.float32)
    for i in range(640 // 16):
        zeros_v[pl.ds(16 * i, 16)] = jnp.zeros((16,), jnp.float32)

    half = _NCHUNK // 2
    lo = c * half + s * _DEG_CPS

    @pl.when(s < _NSUB - 1)
    def _():
        pltpu.sync_copy(zeros_v.at[pl.ds(0, _ROWS_MAIN)],
                        acc.at[pl.ds(s * _ROWS_MAIN, _ROWS_MAIN)])

    @pl.when(s == _NSUB - 1)
    def _():
        pltpu.sync_copy(zeros_v.at[pl.ds(0, _ROWS_LAST)],
                        acc.at[pl.ds((_NSUB - 1) * _ROWS_MAIN, _ROWS_LAST)])

    plsc.subcore_barrier()

    def body(j, carry):
        r = lax.rem(j, 16)

        @pl.when(r == 0)
        def _():
            pltpu.sync_copy(dst2d.at[pl.ds(pl.multiple_of(lo + j, 16), 16)], didx)

        pltpu.sync_copy(ones_v, acc.at[didx.at[r]], add=True)
        return carry

    lax.fori_loop(0, _DEG_CPS, body, 0)
    plsc.subcore_barrier()

    def wout(out):
        @pl.when(s < _NSUB - 1)
        def _():
            pltpu.sync_copy(acc.at[pl.ds(s * _ROWS_MAIN, _ROWS_MAIN)],
                            zeros_v.at[pl.ds(0, _ROWS_MAIN)])
            pltpu.sync_copy(zeros_v.at[pl.ds(0, _ROWS_MAIN)],
                            out.at[pl.ds(s * _ROWS_MAIN, _ROWS_MAIN)])

        @pl.when(s == _NSUB - 1)
        def _():
            pltpu.sync_copy(acc.at[pl.ds((_NSUB - 1) * _ROWS_MAIN, _ROWS_LAST)],
                            zeros_v.at[pl.ds(0, _ROWS_LAST)])
            pltpu.sync_copy(zeros_v.at[pl.ds(0, _ROWS_LAST)],
                            out.at[pl.ds((_NSUB - 1) * _ROWS_MAIN, _ROWS_LAST)])

    @pl.when(c == 0)
    def _():
        wout(out0)

    @pl.when(c == 1)
    def _():
        wout(out1)


_CH = 128
_MP_CPS = _NCHUNK // 2 // _NSUB
_NBUF = 2
_PIECE = 16


@functools.partial(
    pl.kernel,
    out_type=(
        jax.ShapeDtypeStruct((_N, _CH), jnp.float32),
        jax.ShapeDtypeStruct((_N, _CH), jnp.float32),
    ),
    mesh=_mesh(),
    scratch_types=[
        pltpu.VMEM((2, _PIECE, _K), jnp.int32),
        pltpu.VMEM((2, _PIECE, _K), jnp.int32),
        pltpu.VMEM((_NBUF, _K, _CH), jnp.float32),
        pltpu.VMEM_SHARED((_NPAD, _CH), jnp.float32),
        pltpu.SemaphoreType.DMA((_NBUF,)),
        pltpu.SemaphoreType.DMA((2,)),
    ],
)
def _mp_kernel(xws, zeros, src2d, dst2d, out0, out1, sidx, didx, rows, acc,
               gsem, psem):
    c = lax.axis_index("c")
    s = lax.axis_index("s")
    lo = c * (_NCHUNK // 2) + s * _MP_CPS

    def staged(src_at, dst_at, base, total):
        off = 0
        while off < total:
            n = min(_K, total - off)
            pltpu.sync_copy(src_at(pl.ds(base + off, n)),
                            rows.at[0, pl.ds(0, n)])
            pltpu.sync_copy(rows.at[0, pl.ds(0, n)],
                            dst_at(pl.ds(base + off, n)))
            off += n

    @pl.when(s < _NSUB - 1)
    def _():
        staged(lambda d: zeros.at[d], lambda d: acc.at[d],
               s * _ROWS_MAIN, _ROWS_MAIN)

    @pl.when(s == _NSUB - 1)
    def _():
        staged(lambda d: zeros.at[d], lambda d: acc.at[d],
               (_NSUB - 1) * _ROWS_MAIN, _ROWS_LAST)

    def load_piece(j0, slot):
        off = pl.multiple_of(lo + j0, _PIECE)
        pltpu.async_copy(src2d.at[pl.ds(off, _PIECE)], sidx.at[slot],
                         psem.at[0])
        pltpu.async_copy(dst2d.at[pl.ds(off, _PIECE)], didx.at[slot],
                         psem.at[1])

    def wait_piece(slot):
        pltpu.make_async_copy(src2d.at[pl.ds(0, _PIECE)], sidx.at[slot],
                              psem.at[0]).wait()
        pltpu.make_async_copy(dst2d.at[pl.ds(0, _PIECE)], didx.at[slot],
                              psem.at[1]).wait()

    load_piece(0, 0)
    wait_piece(0)
    plsc.subcore_barrier()

    def prime(j, carry):
        pltpu.async_copy(xws.at[sidx.at[0, j]], rows.at[lax.rem(j, _NBUF)],
                         gsem.at[lax.rem(j, _NBUF)])
        return carry

    lax.fori_loop(0, _NBUF, prime, 0)

    def body(j, carry):
        r = lax.rem(j, _PIECE)
        slot = lax.rem(lax.div(j, _PIECE), 2)

        @pl.when(jnp.logical_and(r == 0, j + _PIECE < _MP_CPS))
        def _():
            load_piece(j + _PIECE, lax.rem(slot + 1, 2))

        @pl.when(jnp.logical_and(r == _PIECE - _NBUF, j + _NBUF < _MP_CPS))
        def _():
            wait_piece(lax.rem(slot + 1, 2))

        b = lax.rem(j, _NBUF)
        pltpu.make_async_copy(xws.at[sidx.at[slot, r]], rows.at[b],
                              gsem.at[b]).wait()
        pltpu.sync_copy(rows.at[b], acc.at[didx.at[slot, r]], add=True)

        @pl.when(j < _MP_CPS - _NBUF)
        def _():
            j2 = j + _NBUF
            r2 = lax.rem(j2, _PIECE)
            slot2 = lax.rem(lax.div(j2, _PIECE), 2)
            pltpu.async_copy(xws.at[sidx.at[slot2, r2]], rows.at[b],
                             gsem.at[b])
        return carry

    lax.fori_loop(0, _MP_CPS, body, 0)
    plsc.subcore_barrier()

    def wout(out):
        @pl.when(s < _NSUB - 1)
        def _():
            staged(lambda d: acc.at[d], lambda d: out.at[d],
                   s * _ROWS_MAIN, _ROWS_MAIN)

        @pl.when(s == _NSUB - 1)
        def _():
            staged(lambda d: acc.at[d], lambda d: out.at[d],
                   (_NSUB - 1) * _ROWS_MAIN, _ROWS_LAST)

    @pl.when(c == 0)
    def _():
        wout(out0)

    @pl.when(c == 1)
    def _():
        wout(out1)


def _t1(x, deg0, deg1):
    def body(x_ref, d0_ref, d1_ref, xs_ref, dis_ref):
        deg = d0_ref[...] + d1_ref[...] + 1.0
        dis = lax.rsqrt(deg)
        xs_ref[...] = x_ref[...] * dis
        dis_ref[...] = dis

    return pl.pallas_call(
        body,
        grid=(_GRID,),
        in_specs=[
            pl.BlockSpec((_BN, 128), lambda i: (i, 0)),
            pl.BlockSpec((_BN, 1), lambda i: (i, 0)),
            pl.BlockSpec((_BN, 1), lambda i: (i, 0)),
        ],
        out_specs=[
            pl.BlockSpec((_BN, 128), lambda i: (i, 0)),
            pl.BlockSpec((_BN, 1), lambda i: (i, 0)),
        ],
        out_shape=[
            jax.ShapeDtypeStruct((_N, 128), jnp.float32),
            jax.ShapeDtypeStruct((_N, 1), jnp.float32),
        ],
    )(x, deg0, deg1)


def _t2(p0, p1, xs, dis, W1, b1, W2):
    def body(p0_ref, p1_ref, xs_ref, dis_ref, w1_ref, b_ref, w2_ref, o_ref):
        d = dis_ref[...]
        ax = (p0_ref[...] + p1_ref[...] + xs_ref[...]) * d
        h = jnp.dot(ax, w1_ref[...], preferred_element_type=jnp.float32)
        h = jnp.maximum(h + b_ref[...], 0.0)
        xw = jnp.dot(h, w2_ref[...], preferred_element_type=jnp.float32)
        o_ref[...] = xw * d

    return pl.pallas_call(
        body,
        grid=(_GRID,),
        in_specs=[pl.BlockSpec((_BN, 128), lambda i: (i, 0))] * 3
        + [
            pl.BlockSpec((_BN, 1), lambda i: (i, 0)),
            pl.BlockSpec((128, 256), lambda i: (0, 0)),
            pl.BlockSpec((1, 256), lambda i: (0, 0)),
            pl.BlockSpec((256, 128), lambda i: (0, 0)),
        ],
        out_specs=pl.BlockSpec((_BN, 128), lambda i: (i, 0)),
        out_shape=jax.ShapeDtypeStruct((_N, 128), jnp.float32),
    )(p0, p1, xs, dis, W1, b1, W2)


def _t3(p0, p1, xws2, dis, b2, Wl, bl):
    def body(p0_ref, p1_ref, x2_ref, dis_ref, b2_ref, w_ref, bl_ref, o_ref):
        d = dis_ref[...]
        h = (p0_ref[...] + p1_ref[...] + x2_ref[...]) * d + b2_ref[...]
        o = jnp.dot(h, w_ref[...], preferred_element_type=jnp.float32) + bl_ref[...]
        o_ref[...] = jnp.maximum(o, 0.0)

    return pl.pallas_call(
        body,
        grid=(_GRID,),
        in_specs=[pl.BlockSpec((_BN, 128), lambda i: (i, 0))] * 3
        + [
            pl.BlockSpec((_BN, 1), lambda i: (i, 0)),
            pl.BlockSpec((1, 128), lambda i: (0, 0)),
            pl.BlockSpec((128, 128), lambda i: (0, 0)),
            pl.BlockSpec((1, 128), lambda i: (0, 0)),
        ],
        out_specs=pl.BlockSpec((_BN, 128), lambda i: (i, 0)),
        out_shape=jax.ShapeDtypeStruct((_N, 128), jnp.float32),
    )(p0, p1, xws2, dis, b2, Wl, bl)


def kernel(x, edge_index, W1, b1, W2, b2, Wl, bl):
    ei = edge_index.astype(jnp.int32)
    pad = _E_PAD - _E
    padsrc = jnp.arange(pad, dtype=jnp.int32) % _N
    paddst = _N + (jnp.arange(pad, dtype=jnp.int32) % (_NPAD - _N))
    src2d = jnp.concatenate([ei[0], padsrc]).reshape(_NCHUNK, _K)
    dst2d = jnp.concatenate([ei[1], paddst]).reshape(_NCHUNK, _K)
    zeros = jnp.zeros((_N, _CH), jnp.float32)

    d0, d1 = _deg_kernel(dst2d)
    xs, dis = _t1(x, d0.reshape(_N, 1), d1.reshape(_N, 1))
    p0, p1 = _mp_kernel(xs, zeros, src2d, dst2d)
    xws2 = _t2(p0, p1, xs, dis, W1, b1.reshape(1, -1), W2)
    q0, q1 = _mp_kernel(xws2, zeros, src2d, dst2d)
    return _t3(q0, q1, xws2, dis, b2.reshape(1, -1), Wl, bl.reshape(1, -1))


# speedup vs baseline: 1.6727x; 1.0280x over previous
import functools

import jax
import jax.numpy as jnp
from jax import lax
from jax.experimental import pallas as pl
from jax.experimental.pallas import tpu as pltpu
from jax.experimental.pallas import tpu_sc as plsc

_N = 10000
_E = 320000
_K = 128
_NSUB = 16
_BN = 1000
_GRID = _N // _BN

_NCHUNK = 2560
_E_PAD = _NCHUNK * _K
_NPAD = _N + 16

_ROWS_MAIN = 632
_ROWS_LAST = _N - (_NSUB - 1) * _ROWS_MAIN

_DEG_CPS = _NCHUNK // 2 // _NSUB
_MP_CPS = _NCHUNK // _NSUB


def _mesh():
    return plsc.VectorSubcoreMesh(core_axis_name="c", subcore_axis_name="s")


@functools.partial(
    pl.kernel,
    out_type=(
        jax.ShapeDtypeStruct((_N,), jnp.float32),
        jax.ShapeDtypeStruct((_N,), jnp.float32),
    ),
    mesh=_mesh(),
    scratch_types=[
        pltpu.VMEM((16, _K), jnp.int32),
        pltpu.VMEM((_K,), jnp.float32),
        pltpu.VMEM((640,), jnp.float32),
        pltpu.VMEM_SHARED((_NPAD,), jnp.float32),
    ],
)
def _deg_kernel(dst2d, out0, out1, didx, ones_v, zeros_v, acc):
    c = lax.axis_index("c")
    s = lax.axis_index("s")
    for i in range(_K // 16):
        ones_v[pl.ds(16 * i, 16)] = jnp.full((16,), 1.0, jnp.float32)
    for i in range(640 // 16):
        zeros_v[pl.ds(16 * i, 16)] = jnp.zeros((16,), jnp.float32)

    half = _NCHUNK // 2
    lo = c * half + s * _DEG_CPS

    @pl.when(s < _NSUB - 1)
    def _():
        pltpu.sync_copy(zeros_v.at[pl.ds(0, _ROWS_MAIN)],
                        acc.at[pl.ds(s * _ROWS_MAIN, _ROWS_MAIN)])

    @pl.when(s == _NSUB - 1)
    def _():
        pltpu.sync_copy(zeros_v.at[pl.ds(0, _ROWS_LAST)],
                        acc.at[pl.ds((_NSUB - 1) * _ROWS_MAIN, _ROWS_LAST)])

    plsc.subcore_barrier()

    def body(j, carry):
        r = lax.rem(j, 16)

        @pl.when(r == 0)
        def _():
            pltpu.sync_copy(dst2d.at[pl.ds(pl.multiple_of(lo + j, 16), 16)], didx)

        pltpu.sync_copy(ones_v, acc.at[didx.at[r]], add=True)
        return carry

    lax.fori_loop(0, _DEG_CPS, body, 0)
    plsc.subcore_barrier()

    def wout(out):
        @pl.when(s < _NSUB - 1)
        def _():
            pltpu.sync_copy(acc.at[pl.ds(s * _ROWS_MAIN, _ROWS_MAIN)],
                            zeros_v.at[pl.ds(0, _ROWS_MAIN)])
            pltpu.sync_copy(zeros_v.at[pl.ds(0, _ROWS_MAIN)],
                            out.at[pl.ds(s * _ROWS_MAIN, _ROWS_MAIN)])

        @pl.when(s == _NSUB - 1)
        def _():
            pltpu.sync_copy(acc.at[pl.ds((_NSUB - 1) * _ROWS_MAIN, _ROWS_LAST)],
                            zeros_v.at[pl.ds(0, _ROWS_LAST)])
            pltpu.sync_copy(zeros_v.at[pl.ds(0, _ROWS_LAST)],
                            out.at[pl.ds((_NSUB - 1) * _ROWS_MAIN, _ROWS_LAST)])

    @pl.when(c == 0)
    def _():
        wout(out0)

    @pl.when(c == 1)
    def _():
        wout(out1)


_CH = 128
_MP_CPS = _NCHUNK // 2 // _NSUB
_NBUF = 2
_PIECE = 16


@functools.partial(
    pl.kernel,
    out_type=(
        jax.ShapeDtypeStruct((_N, _CH), jnp.float32),
        jax.ShapeDtypeStruct((_N, _CH), jnp.float32),
    ),
    mesh=_mesh(),
    scratch_types=[
        pltpu.VMEM((2, _PIECE, _K), jnp.int32),
        pltpu.VMEM((2, _PIECE, _K), jnp.int32),
        pltpu.VMEM((_NBUF, _K, _CH), jnp.float32),
        pltpu.VMEM_SHARED((_NPAD, _CH), jnp.float32),
        pltpu.SemaphoreType.DMA((_NBUF,)),
        pltpu.SemaphoreType.DMA((2,)),
    ],
)
def _mp_kernel(xws, zeros, src2d, dst2d, out0, out1, sidx, didx, rows, acc,
               gsem, psem):
    c = lax.axis_index("c")
    s = lax.axis_index("s")
    lo = c * (_NCHUNK // 2) + s * _MP_CPS

    def staged(src_at, dst_at, base, total):
        pieces, off = [], 0
        while off < total:
            n = min(_K, total - off)
            pieces.append((off, n))
            off += n

        def load(i):
            o, n = pieces[i]
            pltpu.async_copy(src_at(pl.ds(base + o, n)),
                             rows.at[i % 2, pl.ds(0, n)], gsem.at[i % 2])

        load(0)
        for i, (o, n) in enumerate(pieces):
            b = i % 2
            pltpu.make_async_copy(src_at(pl.ds(base + o, n)),
                                  rows.at[b, pl.ds(0, n)],
                                  gsem.at[b]).wait()
            if i + 1 < len(pieces):
                load(i + 1)
            pltpu.sync_copy(rows.at[b, pl.ds(0, n)],
                            dst_at(pl.ds(base + o, n)))

    @pl.when(s < _NSUB - 1)
    def _():
        staged(lambda d: zeros.at[d], lambda d: acc.at[d],
               s * _ROWS_MAIN, _ROWS_MAIN)

    @pl.when(s == _NSUB - 1)
    def _():
        staged(lambda d: zeros.at[d], lambda d: acc.at[d],
               (_NSUB - 1) * _ROWS_MAIN, _ROWS_LAST)

    def load_piece(j0, slot):
        off = pl.multiple_of(lo + j0, _PIECE)
        pltpu.async_copy(src2d.at[pl.ds(off, _PIECE)], sidx.at[slot],
                         psem.at[0])
        pltpu.async_copy(dst2d.at[pl.ds(off, _PIECE)], didx.at[slot],
                         psem.at[1])

    def wait_piece(slot):
        pltpu.make_async_copy(src2d.at[pl.ds(0, _PIECE)], sidx.at[slot],
                              psem.at[0]).wait()
        pltpu.make_async_copy(dst2d.at[pl.ds(0, _PIECE)], didx.at[slot],
                              psem.at[1]).wait()

    load_piece(0, 0)
    wait_piece(0)
    plsc.subcore_barrier()

    def prime(j, carry):
        pltpu.async_copy(xws.at[sidx.at[0, j]], rows.at[lax.rem(j, _NBUF)],
                         gsem.at[lax.rem(j, _NBUF)])
        return carry

    lax.fori_loop(0, _NBUF, prime, 0)

    def body(j, carry):
        r = lax.rem(j, _PIECE)
        slot = lax.rem(lax.div(j, _PIECE), 2)

        @pl.when(jnp.logical_and(r == 0, j + _PIECE < _MP_CPS))
        def _():
            load_piece(j + _PIECE, lax.rem(slot + 1, 2))

        @pl.when(jnp.logical_and(r == _PIECE - _NBUF, j + _NBUF < _MP_CPS))
        def _():
            wait_piece(lax.rem(slot + 1, 2))

        b = lax.rem(j, _NBUF)
        pltpu.make_async_copy(xws.at[sidx.at[slot, r]], rows.at[b],
                              gsem.at[b]).wait()
        pltpu.sync_copy(rows.at[b], acc.at[didx.at[slot, r]], add=True)

        @pl.when(j < _MP_CPS - _NBUF)
        def _():
            j2 = j + _NBUF
            r2 = lax.rem(j2, _PIECE)
            slot2 = lax.rem(lax.div(j2, _PIECE), 2)
            pltpu.async_copy(xws.at[sidx.at[slot2, r2]], rows.at[b],
                             gsem.at[b])
        return carry

    lax.fori_loop(0, _MP_CPS, body, 0)
    plsc.subcore_barrier()

    def wout(out):
        @pl.when(s < _NSUB - 1)
        def _():
            staged(lambda d: acc.at[d], lambda d: out.at[d],
                   s * _ROWS_MAIN, _ROWS_MAIN)

        @pl.when(s == _NSUB - 1)
        def _():
            staged(lambda d: acc.at[d], lambda d: out.at[d],
                   (_NSUB - 1) * _ROWS_MAIN, _ROWS_LAST)

    @pl.when(c == 0)
    def _():
        wout(out0)

    @pl.when(c == 1)
    def _():
        wout(out1)


def _t1(x, deg0, deg1):
    def body(x_ref, d0_ref, d1_ref, xs_ref, dis_ref):
        deg = d0_ref[...] + d1_ref[...] + 1.0
        dis = lax.rsqrt(deg)
        xs_ref[...] = x_ref[...] * dis
        dis_ref[...] = dis

    return pl.pallas_call(
        body,
        grid=(_GRID,),
        in_specs=[
            pl.BlockSpec((_BN, 128), lambda i: (i, 0)),
            pl.BlockSpec((_BN, 1), lambda i: (i, 0)),
            pl.BlockSpec((_BN, 1), lambda i: (i, 0)),
        ],
        out_specs=[
            pl.BlockSpec((_BN, 128), lambda i: (i, 0)),
            pl.BlockSpec((_BN, 1), lambda i: (i, 0)),
        ],
        out_shape=[
            jax.ShapeDtypeStruct((_N, 128), jnp.float32),
            jax.ShapeDtypeStruct((_N, 1), jnp.float32),
        ],
    )(x, deg0, deg1)


def _t2(p0, p1, xs, dis, W1, b1, W2):
    def body(p0_ref, p1_ref, xs_ref, dis_ref, w1_ref, b_ref, w2_ref, o_ref):
        d = dis_ref[...]
        ax = (p0_ref[...] + p1_ref[...] + xs_ref[...]) * d
        h = jnp.dot(ax, w1_ref[...], preferred_element_type=jnp.float32)
        h = jnp.maximum(h + b_ref[...], 0.0)
        xw = jnp.dot(h, w2_ref[...], preferred_element_type=jnp.float32)
        o_ref[...] = xw * d

    return pl.pallas_call(
        body,
        grid=(_GRID,),
        in_specs=[pl.BlockSpec((_BN, 128), lambda i: (i, 0))] * 3
        + [
            pl.BlockSpec((_BN, 1), lambda i: (i, 0)),
            pl.BlockSpec((128, 256), lambda i: (0, 0)),
            pl.BlockSpec((1, 256), lambda i: (0, 0)),
            pl.BlockSpec((256, 128), lambda i: (0, 0)),
        ],
        out_specs=pl.BlockSpec((_BN, 128), lambda i: (i, 0)),
        out_shape=jax.ShapeDtypeStruct((_N, 128), jnp.float32),
    )(p0, p1, xs, dis, W1, b1, W2)


def _t3(p0, p1, xws2, dis, b2, Wl, bl):
    def body(p0_ref, p1_ref, x2_ref, dis_ref, b2_ref, w_ref, bl_ref, o_ref):
        d = dis_ref[...]
        h = (p0_ref[...] + p1_ref[...] + x2_ref[...]) * d + b2_ref[...]
        o = jnp.dot(h, w_ref[...], preferred_element_type=jnp.float32) + bl_ref[...]
        o_ref[...] = jnp.maximum(o, 0.0)

    return pl.pallas_call(
        body,
        grid=(_GRID,),
        in_specs=[pl.BlockSpec((_BN, 128), lambda i: (i, 0))] * 3
        + [
            pl.BlockSpec((_BN, 1), lambda i: (i, 0)),
            pl.BlockSpec((1, 128), lambda i: (0, 0)),
            pl.BlockSpec((128, 128), lambda i: (0, 0)),
            pl.BlockSpec((1, 128), lambda i: (0, 0)),
        ],
        out_specs=pl.BlockSpec((_BN, 128), lambda i: (i, 0)),
        out_shape=jax.ShapeDtypeStruct((_N, 128), jnp.float32),
    )(p0, p1, xws2, dis, b2, Wl, bl)


def kernel(x, edge_index, W1, b1, W2, b2, Wl, bl):
    ei = edge_index.astype(jnp.int32)
    pad = _E_PAD - _E
    padsrc = jnp.arange(pad, dtype=jnp.int32) % _N
    paddst = _N + (jnp.arange(pad, dtype=jnp.int32) % (_NPAD - _N))
    src2d = jnp.concatenate([ei[0], padsrc]).reshape(_NCHUNK, _K)
    dst2d = jnp.concatenate([ei[1], paddst]).reshape(_NCHUNK, _K)
    zeros = jnp.zeros((_N, _CH), jnp.float32)

    d0, d1 = _deg_kernel(dst2d)
    xs, dis = _t1(x, d0.reshape(_N, 1), d1.reshape(_N, 1))
    p0, p1 = _mp_kernel(xs, zeros, src2d, dst2d)
    xws2 = _t2(p0, p1, xs, dis, W1, b1.reshape(1, -1), W2)
    q0, q1 = _mp_kernel(xws2, zeros, src2d, dst2d)
    return _t3(q0, q1, xws2, dis, b2.reshape(1, -1), Wl, bl.reshape(1, -1))


# speedup vs baseline: 1.6939x; 1.0126x over previous
import functools

import jax
import jax.numpy as jnp
from jax import lax
from jax.experimental import pallas as pl
from jax.experimental.pallas import tpu as pltpu
from jax.experimental.pallas import tpu_sc as plsc

_N = 10000
_E = 320000
_K = 128
_NSUB = 16
_BN = 1000
_GRID = _N // _BN

_NCHUNK = 2560
_E_PAD = _NCHUNK * _K
_NPAD = _N + 16

_ROWS_MAIN = 632
_ROWS_LAST = _N - (_NSUB - 1) * _ROWS_MAIN

_DEG_CPS = _NCHUNK // 2 // _NSUB
_MP_CPS = _NCHUNK // _NSUB


def _mesh():
    return plsc.VectorSubcoreMesh(core_axis_name="c", subcore_axis_name="s")


@functools.partial(
    pl.kernel,
    out_type=(
        jax.ShapeDtypeStruct((_N,), jnp.float32),
        jax.ShapeDtypeStruct((_N,), jnp.float32),
    ),
    mesh=_mesh(),
    scratch_types=[
        pltpu.VMEM((3, 16, _K), jnp.int32),
        pltpu.VMEM((_K,), jnp.float32),
        pltpu.VMEM((640,), jnp.float32),
        pltpu.VMEM_SHARED((_NPAD,), jnp.float32),
        pltpu.SemaphoreType.DMA((2,)),
    ],
)
def _deg_kernel(dst2d, out0, out1, didx, ones_v, zeros_v, acc, ssem):
    c = lax.axis_index("c")
    s = lax.axis_index("s")
    for i in range(_K // 16):
        ones_v[pl.ds(16 * i, 16)] = jnp.full((16,), 1.0, jnp.float32)
    for i in range(640 // 16):
        zeros_v[pl.ds(16 * i, 16)] = jnp.zeros((16,), jnp.float32)

    half = _NCHUNK // 2
    lo = c * half + s * _DEG_CPS

    @pl.when(s < _NSUB - 1)
    def _():
        pltpu.sync_copy(zeros_v.at[pl.ds(0, _ROWS_MAIN)],
                        acc.at[pl.ds(s * _ROWS_MAIN, _ROWS_MAIN)])

    @pl.when(s == _NSUB - 1)
    def _():
        pltpu.sync_copy(zeros_v.at[pl.ds(0, _ROWS_LAST)],
                        acc.at[pl.ds((_NSUB - 1) * _ROWS_MAIN, _ROWS_LAST)])

    plsc.subcore_barrier()

    pltpu.sync_copy(dst2d.at[pl.ds(pl.multiple_of(lo, 16), 16)], didx.at[0])

    def body(j, carry):
        r = lax.rem(j, 16)
        slot = lax.rem(lax.div(j, 16), 3)

        @pl.when(jnp.logical_and(r == 0, j + 16 < _DEG_CPS))
        def _():
            off = pl.multiple_of(lo + j + 16, 16)
            pltpu.sync_copy(dst2d.at[pl.ds(off, 16)],
                            didx.at[lax.rem(slot + 1, 3)])

        @pl.when(j >= 2)
        def _():
            pltpu.make_async_copy(ones_v, acc.at[didx.at[0, 0]],
                                  ssem.at[lax.rem(j, 2)]).wait()

        pltpu.async_copy(ones_v, acc.at[didx.at[slot, r]],
                         ssem.at[lax.rem(j, 2)], add=True)
        return carry

    lax.fori_loop(0, _DEG_CPS, body, 0)
    for b in range(2):
        pltpu.make_async_copy(ones_v, acc.at[didx.at[0, 0]],
                              ssem.at[b]).wait()
    plsc.subcore_barrier()

    def wout(out):
        @pl.when(s < _NSUB - 1)
        def _():
            pltpu.sync_copy(acc.at[pl.ds(s * _ROWS_MAIN, _ROWS_MAIN)],
                            zeros_v.at[pl.ds(0, _ROWS_MAIN)])
            pltpu.sync_copy(zeros_v.at[pl.ds(0, _ROWS_MAIN)],
                            out.at[pl.ds(s * _ROWS_MAIN, _ROWS_MAIN)])

        @pl.when(s == _NSUB - 1)
        def _():
            pltpu.sync_copy(acc.at[pl.ds((_NSUB - 1) * _ROWS_MAIN, _ROWS_LAST)],
                            zeros_v.at[pl.ds(0, _ROWS_LAST)])
            pltpu.sync_copy(zeros_v.at[pl.ds(0, _ROWS_LAST)],
                            out.at[pl.ds((_NSUB - 1) * _ROWS_MAIN, _ROWS_LAST)])

    @pl.when(c == 0)
    def _():
        wout(out0)

    @pl.when(c == 1)
    def _():
        wout(out1)


_CH = 128
_MP_CPS = _NCHUNK // 2 // _NSUB
_NBUF = 2
_PIECE = 16


@functools.partial(
    pl.kernel,
    out_type=(
        jax.ShapeDtypeStruct((_N, _CH), jnp.float32),
        jax.ShapeDtypeStruct((_N, _CH), jnp.float32),
    ),
    mesh=_mesh(),
    scratch_types=[
        pltpu.VMEM((2, _PIECE, _K), jnp.int32),
        pltpu.VMEM((2, _PIECE, _K), jnp.int32),
        pltpu.VMEM((_NBUF, _K, _CH), jnp.float32),
        pltpu.VMEM_SHARED((_NPAD, _CH), jnp.float32),
        pltpu.SemaphoreType.DMA((_NBUF,)),
        pltpu.SemaphoreType.DMA((2,)),
    ],
)
def _mp_kernel(xws, zeros, src2d, dst2d, out0, out1, sidx, didx, rows, acc,
               gsem, psem):
    c = lax.axis_index("c")
    s = lax.axis_index("s")
    lo = c * (_NCHUNK // 2) + s * _MP_CPS

    def staged(src_at, dst_at, base, total):
        pieces, off = [], 0
        while off < total:
            n = min(_K, total - off)
            pieces.append((off, n))
            off += n

        def load(i):
            o, n = pieces[i]
            pltpu.async_copy(src_at(pl.ds(base + o, n)),
                             rows.at[i % 2, pl.ds(0, n)], gsem.at[i % 2])

        load(0)
        for i, (o, n) in enumerate(pieces):
            b = i % 2
            pltpu.make_async_copy(src_at(pl.ds(base + o, n)),
                                  rows.at[b, pl.ds(0, n)],
                                  gsem.at[b]).wait()
            if i + 1 < len(pieces):
                load(i + 1)
            pltpu.sync_copy(rows.at[b, pl.ds(0, n)],
                            dst_at(pl.ds(base + o, n)))

    @pl.when(s < _NSUB - 1)
    def _():
        staged(lambda d: zeros.at[d], lambda d: acc.at[d],
               s * _ROWS_MAIN, _ROWS_MAIN)

    @pl.when(s == _NSUB - 1)
    def _():
        staged(lambda d: zeros.at[d], lambda d: acc.at[d],
               (_NSUB - 1) * _ROWS_MAIN, _ROWS_LAST)

    def load_piece(j0, slot):
        off = pl.multiple_of(lo + j0, _PIECE)
        pltpu.async_copy(src2d.at[pl.ds(off, _PIECE)], sidx.at[slot],
                         psem.at[0])
        pltpu.async_copy(dst2d.at[pl.ds(off, _PIECE)], didx.at[slot],
                         psem.at[1])

    def wait_piece(slot):
        pltpu.make_async_copy(src2d.at[pl.ds(0, _PIECE)], sidx.at[slot],
                              psem.at[0]).wait()
        pltpu.make_async_copy(dst2d.at[pl.ds(0, _PIECE)], didx.at[slot],
                              psem.at[1]).wait()

    load_piece(0, 0)
    wait_piece(0)
    plsc.subcore_barrier()

    def prime(j, carry):
        pltpu.async_copy(xws.at[sidx.at[0, j]], rows.at[lax.rem(j, _NBUF)],
                         gsem.at[lax.rem(j, _NBUF)])
        return carry

    lax.fori_loop(0, _NBUF, prime, 0)

    def body(j, carry):
        r = lax.rem(j, _PIECE)
        slot = lax.rem(lax.div(j, _PIECE), 2)

        @pl.when(jnp.logical_and(r == 0, j + _PIECE < _MP_CPS))
        def _():
            load_piece(j + _PIECE, lax.rem(slot + 1, 2))

        @pl.when(jnp.logical_and(r == _PIECE - _NBUF, j + _NBUF < _MP_CPS))
        def _():
            wait_piece(lax.rem(slot + 1, 2))

        b = lax.rem(j, _NBUF)
        pltpu.make_async_copy(xws.at[sidx.at[slot, r]], rows.at[b],
                              gsem.at[b]).wait()
        pltpu.sync_copy(rows.at[b], acc.at[didx.at[slot, r]], add=True)

        @pl.when(j < _MP_CPS - _NBUF)
        def _():
            j2 = j + _NBUF
            r2 = lax.rem(j2, _PIECE)
            slot2 = lax.rem(lax.div(j2, _PIECE), 2)
            pltpu.async_copy(xws.at[sidx.at[slot2, r2]], rows.at[b],
                             gsem.at[b])
        return carry

    lax.fori_loop(0, _MP_CPS, body, 0)
    plsc.subcore_barrier()

    def wout(out):
        @pl.when(s < _NSUB - 1)
        def _():
            staged(lambda d: acc.at[d], lambda d: out.at[d],
                   s * _ROWS_MAIN, _ROWS_MAIN)

        @pl.when(s == _NSUB - 1)
        def _():
            staged(lambda d: acc.at[d], lambda d: out.at[d],
                   (_NSUB - 1) * _ROWS_MAIN, _ROWS_LAST)

    @pl.when(c == 0)
    def _():
        wout(out0)

    @pl.when(c == 1)
    def _():
        wout(out1)


def _t1(x, deg0, deg1):
    def body(x_ref, d0_ref, d1_ref, xs_ref, dis_ref):
        deg = d0_ref[...] + d1_ref[...] + 1.0
        dis = lax.rsqrt(deg)
        xs_ref[...] = x_ref[...] * dis
        dis_ref[...] = dis

    return pl.pallas_call(
        body,
        grid=(_GRID,),
        in_specs=[
            pl.BlockSpec((_BN, 128), lambda i: (i, 0)),
            pl.BlockSpec((_BN, 1), lambda i: (i, 0)),
            pl.BlockSpec((_BN, 1), lambda i: (i, 0)),
        ],
        out_specs=[
            pl.BlockSpec((_BN, 128), lambda i: (i, 0)),
            pl.BlockSpec((_BN, 1), lambda i: (i, 0)),
        ],
        out_shape=[
            jax.ShapeDtypeStruct((_N, 128), jnp.float32),
            jax.ShapeDtypeStruct((_N, 1), jnp.float32),
        ],
    )(x, deg0, deg1)


def _t2(p0, p1, xs, dis, W1, b1, W2):
    def body(p0_ref, p1_ref, xs_ref, dis_ref, w1_ref, b_ref, w2_ref, o_ref):
        d = dis_ref[...]
        ax = (p0_ref[...] + p1_ref[...] + xs_ref[...]) * d
        h = jnp.dot(ax, w1_ref[...], preferred_element_type=jnp.float32)
        h = jnp.maximum(h + b_ref[...], 0.0)
        xw = jnp.dot(h, w2_ref[...], preferred_element_type=jnp.float32)
        o_ref[...] = xw * d

    return pl.pallas_call(
        body,
        grid=(_GRID,),
        in_specs=[pl.BlockSpec((_BN, 128), lambda i: (i, 0))] * 3
        + [
            pl.BlockSpec((_BN, 1), lambda i: (i, 0)),
            pl.BlockSpec((128, 256), lambda i: (0, 0)),
            pl.BlockSpec((1, 256), lambda i: (0, 0)),
            pl.BlockSpec((256, 128), lambda i: (0, 0)),
        ],
        out_specs=pl.BlockSpec((_BN, 128), lambda i: (i, 0)),
        out_shape=jax.ShapeDtypeStruct((_N, 128), jnp.float32),
    )(p0, p1, xs, dis, W1, b1, W2)


def _t3(p0, p1, xws2, dis, b2, Wl, bl):
    def body(p0_ref, p1_ref, x2_ref, dis_ref, b2_ref, w_ref, bl_ref, o_ref):
        d = dis_ref[...]
        h = (p0_ref[...] + p1_ref[...] + x2_ref[...]) * d + b2_ref[...]
        o = jnp.dot(h, w_ref[...], preferred_element_type=jnp.float32) + bl_ref[...]
        o_ref[...] = jnp.maximum(o, 0.0)

    return pl.pallas_call(
        body,
        grid=(_GRID,),
        in_specs=[pl.BlockSpec((_BN, 128), lambda i: (i, 0))] * 3
        + [
            pl.BlockSpec((_BN, 1), lambda i: (i, 0)),
            pl.BlockSpec((1, 128), lambda i: (0, 0)),
            pl.BlockSpec((128, 128), lambda i: (0, 0)),
            pl.BlockSpec((1, 128), lambda i: (0, 0)),
        ],
        out_specs=pl.BlockSpec((_BN, 128), lambda i: (i, 0)),
        out_shape=jax.ShapeDtypeStruct((_N, 128), jnp.float32),
    )(p0, p1, xws2, dis, b2, Wl, bl)


def kernel(x, edge_index, W1, b1, W2, b2, Wl, bl):
    ei = edge_index.astype(jnp.int32)
    pad = _E_PAD - _E
    padsrc = jnp.arange(pad, dtype=jnp.int32) % _N
    paddst = _N + (jnp.arange(pad, dtype=jnp.int32) % (_NPAD - _N))
    src2d = jnp.concatenate([ei[0], padsrc]).reshape(_NCHUNK, _K)
    dst2d = jnp.concatenate([ei[1], paddst]).reshape(_NCHUNK, _K)
    zeros = jnp.zeros((_N, _CH), jnp.float32)

    d0, d1 = _deg_kernel(dst2d)
    xs, dis = _t1(x, d0.reshape(_N, 1), d1.reshape(_N, 1))
    p0, p1 = _mp_kernel(xs, zeros, src2d, dst2d)
    xws2 = _t2(p0, p1, xs, dis, W1, b1.reshape(1, -1), W2)
    q0, q1 = _mp_kernel(xws2, zeros, src2d, dst2d)
    return _t3(q0, q1, xws2, dis, b2.reshape(1, -1), Wl, bl.reshape(1, -1))
